# baseline (speedup 1.0000x reference)
import functools

import jax
import jax.numpy as jnp
from jax import lax
from jax.experimental import pallas as pl
from jax.experimental.pallas import tpu as pltpu
from jax.experimental.pallas import tpu_sc as plsc

N_NODES = 10000
N_PAD = 10240
FDIM = 128
NCLS = 40
NGRAPH = 64
N_EDGES = 320000
NCORES = 2
NSUB = 16
NW = NCORES * NSUB
CHUNK = 128
TOTC = 2560
E_PAD = TOTC * CHUNK
NC0 = 156
NC1 = 4
JUNK_ROW = N_PAD - 1
RPT = N_PAD // NSUB
BM = 1024
NBLK = N_PAD // BM

_mesh = plsc.VectorSubcoreMesh(core_axis_name="c", subcore_axis_name="s",
                               num_cores=NCORES)


def _fill2(ref, nrows, ncols, val):
    v = jnp.full((16,), val, jnp.float32)

    def row(i, carry):
        for k in range(ncols // 16):
            ref[i, pl.ds(k * 16, 16)] = v
        return carry

    lax.fori_loop(0, nrows, row, 0)


def _fill1(ref, n, val):
    v = jnp.full((16,), val, jnp.float32)

    def step(i, carry):
        ref[pl.ds(i * 16, 16)] = v
        return carry

    lax.fori_loop(0, n // 16, step, 0)


def _sc_agg_body(with_deg, table, eir, *rest):
    if with_deg:
        (out, dout, acc, degs, ib0, ib1, rows0, rows1, ones_v, zv,
         g0, g1, s0, s1, d0, d1) = rest
    else:
        (out, acc, ib0, ib1, rows0, rows1, g0, g1, s0, s1) = rest
        degs = dout = ones_v = zv = d0 = d1 = None
    ibufs = (ib0, ib1)
    rowss = (rows0, rows1)
    gsem = (g0, g1)
    ssem = (s0, s1)
    dsem = (d0, d1)
    c = lax.axis_index("c")
    s = lax.axis_index("s")
    base = s * RPT
    nchunk = jnp.where(c == 0, NC0, NC1)
    k0 = jnp.where(c == 0, s * NC0, NSUB * NC0 + s * NC1)

    def load_idx(j, b):
        pltpu.sync_copy(eir.at[pl.ds(2 * (k0 + j), 2)], ibufs[b])

    def gather_start(b):
        pltpu.async_copy(table.at[ibufs[b].at[0]], rowss[b], gsem[b])

    def gather_wait(b):
        pltpu.make_async_copy(
            table.at[ibufs[b].at[0]], rowss[b], gsem[b]).wait()

    def scatter_start(b):
        pltpu.async_copy(rowss[b], acc.at[ibufs[b].at[1]], ssem[b], add=True)
        if with_deg:
            pltpu.async_copy(ones_v, degs.at[ibufs[b].at[1]], dsem[b],
                             add=True)

    def scatter_wait(b):
        pltpu.make_async_copy(rowss[b], acc.at[ibufs[b].at[1]], ssem[b]).wait()
        if with_deg:
            pltpu.make_async_copy(
                ones_v, degs.at[ibufs[b].at[1]], dsem[b]).wait()

    _fill2(rows0, CHUNK, FDIM, 0.0)
    for k in range(RPT // CHUNK):
        pltpu.sync_copy(rows0, acc.at[pl.ds(base + k * CHUNK, CHUNK)])
    if with_deg:
        _fill1(zv, RPT, 0.0)
        pltpu.sync_copy(zv, degs.at[pl.ds(base, RPT)])
        _fill1(ones_v, CHUNK, 1.0)
    load_idx(0, 0)
    gather_start(0)
    load_idx(1, 1)
    gather_start(1)
    plsc.subcore_barrier()
    gather_wait(0)
    scatter_start(0)

    def step(j, cur):
        nxt = 1 - cur
        scatter_wait(nxt)
        load_idx(j + 1, nxt)
        gather_start(nxt)
        gather_wait(cur)
        scatter_start(cur)

    def pair(p, carry):
        step(2 * p + 1, 1)
        step(2 * p + 2, 0)
        return carry

    lax.fori_loop(0, (nchunk - 2) // 2, pair, 0)
    scatter_wait(0)
    gather_wait(1)
    scatter_start(1)
    scatter_wait(1)
    plsc.subcore_barrier()
    pltpu.sync_copy(acc.at[pl.ds(base, RPT)], out.at[c, pl.ds(base, RPT)])
    if with_deg:
        pltpu.sync_copy(degs.at[pl.ds(base, RPT)],
                        dout.at[pl.ds(c * N_PAD + base, RPT)])


_agg_deg = pl.kernel(
    functools.partial(_sc_agg_body, True),
    out_type=[
        jax.ShapeDtypeStruct((NCORES, N_PAD, FDIM), jnp.float32),
        jax.ShapeDtypeStruct((NCORES * N_PAD,), jnp.float32),
    ],
    mesh=_mesh,
    scratch_types=[
        pltpu.VMEM_SHARED((N_PAD, FDIM), jnp.float32),
        pltpu.VMEM_SHARED((N_PAD,), jnp.float32),
        pltpu.VMEM((2, CHUNK), jnp.int32),
        pltpu.VMEM((2, CHUNK), jnp.int32),
        pltpu.VMEM((CHUNK, FDIM), jnp.float32),
        pltpu.VMEM((CHUNK, FDIM), jnp.float32),
        pltpu.VMEM((CHUNK,), jnp.float32),
        pltpu.VMEM((RPT,), jnp.float32),
        pltpu.SemaphoreType.DMA,
        pltpu.SemaphoreType.DMA,
        pltpu.SemaphoreType.DMA,
        pltpu.SemaphoreType.DMA,
        pltpu.SemaphoreType.DMA,
        pltpu.SemaphoreType.DMA,
    ],
)

_agg = pl.kernel(
    functools.partial(_sc_agg_body, False),
    out_type=[jax.ShapeDtypeStruct((NCORES, N_PAD, FDIM), jnp.float32)],
    mesh=_mesh,
    scratch_types=[
        pltpu.VMEM_SHARED((N_PAD, FDIM), jnp.float32),
        pltpu.VMEM((2, CHUNK), jnp.int32),
        pltpu.VMEM((2, CHUNK), jnp.int32),
        pltpu.VMEM((CHUNK, FDIM), jnp.float32),
        pltpu.VMEM((CHUNK, FDIM), jnp.float32),
        pltpu.SemaphoreType.DMA,
        pltpu.SemaphoreType.DMA,
        pltpu.SemaphoreType.DMA,
        pltpu.SemaphoreType.DMA,
    ],
)


def _dinv_from(pdeg_ref):
    d = pdeg_ref[0, 0]
    for k in range(1, NCORES):
        d = d + pdeg_ref[k, 0]
    return 1.0 / jnp.maximum(d, 1.0)


def _psum(p_ref):
    agg = p_ref[0]
    for k in range(1, NCORES):
        agg = agg + p_ref[k]
    return agg


def _c1_body(p_ref, pdeg_ref, w_ref, b_ref, out_ref):
    agg = _psum(p_ref) * _dinv_from(pdeg_ref)
    h = jnp.dot(agg, w_ref[...], preferred_element_type=jnp.float32) + b_ref[...]
    out_ref[...] = jnp.maximum(h, 0.0)


def _c2_body(p_ref, pdeg_ref, w_ref, b_ref, wout_ref, bout_ref, batch_ref,
             out_ref, s_acc, c_acc):
    i = pl.program_id(0)

    @pl.when(i == 0)
    def _():
        s_acc[...] = jnp.zeros_like(s_acc)
        c_acc[...] = jnp.zeros_like(c_acc)

    agg = _psum(p_ref) * _dinv_from(pdeg_ref)
    h2 = jnp.maximum(
        jnp.dot(agg, w_ref[...], preferred_element_type=jnp.float32) + b_ref[...],
        0.0)
    bb = batch_ref[0]
    gids = lax.broadcasted_iota(jnp.int32, (NGRAPH, BM), 0)
    oh = jnp.where(gids == bb, 1.0, 0.0)
    s_acc[...] += jnp.dot(oh, h2, preferred_element_type=jnp.float32)
    c_acc[...] += jnp.broadcast_to(
        jnp.sum(oh, axis=1, keepdims=True), (NGRAPH, FDIM))

    @pl.when(i == pl.num_programs(0) - 1)
    def _():
        cnt = c_acc[:, 0:NCLS]
        num = (jnp.dot(s_acc[...], wout_ref[...],
                       preferred_element_type=jnp.float32)
               + cnt * bout_ref[...])
        out_ref[...] = num / jnp.maximum(cnt, 1.0)


def _c1(p, pdeg, W, br):
    return pl.pallas_call(
        _c1_body,
        grid=(NBLK,),
        in_specs=[
            pl.BlockSpec((NCORES, BM, FDIM), lambda i: (0, i, 0)),
            pl.BlockSpec((NCORES, 1, BM, 1), lambda i: (0, i, 0, 0)),
            pl.BlockSpec((FDIM, FDIM), lambda i: (0, 0)),
            pl.BlockSpec((1, FDIM), lambda i: (0, 0)),
        ],
        out_specs=pl.BlockSpec((BM, FDIM), lambda i: (i, 0)),
        out_shape=jax.ShapeDtypeStruct((N_PAD, FDIM), jnp.float32),
    )(p, pdeg, W, br)


def _c2(p, pdeg, W, br, Wout, boutr, batch_r):
    return pl.pallas_call(
        _c2_body,
        grid=(NBLK,),
        in_specs=[
            pl.BlockSpec((NCORES, BM, FDIM), lambda i: (0, i, 0)),
            pl.BlockSpec((NCORES, 1, BM, 1), lambda i: (0, i, 0, 0)),
            pl.BlockSpec((FDIM, FDIM), lambda i: (0, 0)),
            pl.BlockSpec((1, FDIM), lambda i: (0, 0)),
            pl.BlockSpec((FDIM, NCLS), lambda i: (0, 0)),
            pl.BlockSpec((1, NCLS), lambda i: (0, 0)),
            pl.BlockSpec((1, 1, BM), lambda i: (i, 0, 0)),
        ],
        out_specs=pl.BlockSpec((NGRAPH, NCLS), lambda i: (0, 0)),
        out_shape=jax.ShapeDtypeStruct((NGRAPH, NCLS), jnp.float32),
        scratch_shapes=[
            pltpu.VMEM((NGRAPH, FDIM), jnp.float32),
            pltpu.VMEM((NGRAPH, FDIM), jnp.float32),
        ],
    )(p, pdeg, W, br, Wout, boutr, batch_r)


def kernel(x, edge_index, batch, W0, b0, W1, b1, Wout, bout):
    src = edge_index[0].astype(jnp.int32)
    dst = edge_index[1].astype(jnp.int32)
    npad = E_PAD - N_EDGES
    src_p = jnp.concatenate(
        [src, jnp.zeros((npad,), jnp.int32)]).reshape(TOTC, CHUNK)
    dst_p = jnp.concatenate(
        [dst, jnp.full((npad,), JUNK_ROW, jnp.int32)]).reshape(TOTC, CHUNK)
    eir = jnp.stack([src_p, dst_p], axis=1).reshape(2 * TOTC, CHUNK)
    batch_p = jnp.concatenate(
        [batch.astype(jnp.int32),
         jnp.full((N_PAD - N_NODES,), NGRAPH, jnp.int32)]).reshape(NBLK, 1, BM)
    p0, deg_flat = _agg_deg(x, eir)
    pdeg = deg_flat.reshape(NCORES, NBLK, BM, 1)
    h1 = _c1(p0, pdeg, W0, b0.reshape(1, FDIM))
    (p1,) = _agg(h1, eir)
    return _c2(p1, pdeg, W1, b1.reshape(1, FDIM), Wout,
               bout.reshape(1, NCLS), batch_p)


# speedup vs baseline: 1.0436x; 1.0436x over previous
import functools

import jax
import jax.numpy as jnp
from jax import lax
from jax.experimental import pallas as pl
from jax.experimental.pallas import tpu as pltpu
from jax.experimental.pallas import tpu_sc as plsc

N_NODES = 10000
N_PAD = 10240
FDIM = 128
NCLS = 40
NGRAPH = 64
N_EDGES = 320000
NCORES = 2
NSUB = 16
NW = NCORES * NSUB
CHUNK = 128
TOTC = 2560
E_PAD = TOTC * CHUNK
NC0 = 152
NC1 = 8
JUNK_ROW = N_PAD - 1
RPT = N_PAD // NSUB
BM = 1024
NBLK = N_PAD // BM

_mesh = plsc.VectorSubcoreMesh(core_axis_name="c", subcore_axis_name="s",
                               num_cores=NCORES)


def _fill2(ref, nrows, ncols, val):
    v = jnp.full((16,), val, jnp.float32)

    def row(i, carry):
        for k in range(ncols // 16):
            ref[i, pl.ds(k * 16, 16)] = v
        return carry

    lax.fori_loop(0, nrows, row, 0)


def _fill1(ref, n, val):
    v = jnp.full((16,), val, jnp.float32)

    def step(i, carry):
        ref[pl.ds(i * 16, 16)] = v
        return carry

    lax.fori_loop(0, n // 16, step, 0)


def _sc_agg_body(with_deg, table, eir, *rest):
    if with_deg:
        (out, dout, acc, degs, ib0, ib1, rows0, rows1, ones_v, zv,
         g0, g1, s0, s1, d0, d1) = rest
    else:
        (out, acc, ib0, ib1, rows0, rows1, g0, g1, s0, s1) = rest
        degs = dout = ones_v = zv = d0 = d1 = None
    ibufs = (ib0, ib1)
    rowss = (rows0, rows1)
    gsem = (g0, g1)
    ssem = (s0, s1)
    dsem = (d0, d1)
    c = lax.axis_index("c")
    s = lax.axis_index("s")
    base = s * RPT
    nchunk = jnp.where(c == 0, NC0, NC1)
    k0 = jnp.where(c == 0, s * NC0, NSUB * NC0 + s * NC1)

    def load_idx(j, b):
        pltpu.sync_copy(eir.at[pl.ds(2 * (k0 + j), 2)], ibufs[b])

    def gather_start(b):
        pltpu.async_copy(table.at[ibufs[b].at[0]], rowss[b], gsem[b])

    def gather_wait(b):
        pltpu.make_async_copy(
            table.at[ibufs[b].at[0]], rowss[b], gsem[b]).wait()

    def scatter_start(b):
        pltpu.async_copy(rowss[b], acc.at[ibufs[b].at[1]], ssem[b], add=True)
        if with_deg:
            pltpu.async_copy(ones_v, degs.at[ibufs[b].at[1]], dsem[b],
                             add=True)

    def scatter_wait(b):
        pltpu.make_async_copy(rowss[b], acc.at[ibufs[b].at[1]], ssem[b]).wait()
        if with_deg:
            pltpu.make_async_copy(
                ones_v, degs.at[ibufs[b].at[1]], dsem[b]).wait()

    _fill2(rows0, CHUNK, FDIM, 0.0)
    for k in range(RPT // CHUNK):
        pltpu.sync_copy(rows0, acc.at[pl.ds(base + k * CHUNK, CHUNK)])
    if with_deg:
        _fill1(zv, RPT, 0.0)
        pltpu.sync_copy(zv, degs.at[pl.ds(base, RPT)])
        _fill1(ones_v, CHUNK, 1.0)
    load_idx(0, 0)
    gather_start(0)
    load_idx(1, 1)
    gather_start(1)
    plsc.subcore_barrier()
    gather_wait(0)
    scatter_start(0)

    def step(j, cur):
        nxt = 1 - cur
        scatter_wait(nxt)
        load_idx(j + 1, nxt)
        gather_start(nxt)
        gather_wait(cur)
        scatter_start(cur)

    def pair(p, carry):
        step(2 * p + 1, 1)
        step(2 * p + 2, 0)
        return carry

    lax.fori_loop(0, (nchunk - 2) // 2, pair, 0)
    scatter_wait(0)
    gather_wait(1)
    scatter_start(1)
    scatter_wait(1)
    plsc.subcore_barrier()
    pltpu.sync_copy(acc.at[pl.ds(base, RPT)], out.at[c, pl.ds(base, RPT)])
    if with_deg:
        pltpu.sync_copy(degs.at[pl.ds(base, RPT)],
                        dout.at[pl.ds(c * N_PAD + base, RPT)])


_agg_deg = pl.kernel(
    functools.partial(_sc_agg_body, True),
    out_type=[
        jax.ShapeDtypeStruct((NCORES, N_PAD, FDIM), jnp.float32),
        jax.ShapeDtypeStruct((NCORES * N_PAD,), jnp.float32),
    ],
    mesh=_mesh,
    scratch_types=[
        pltpu.VMEM_SHARED((N_PAD, FDIM), jnp.float32),
        pltpu.VMEM_SHARED((N_PAD,), jnp.float32),
        pltpu.VMEM((2, CHUNK), jnp.int32),
        pltpu.VMEM((2, CHUNK), jnp.int32),
        pltpu.VMEM((CHUNK, FDIM), jnp.float32),
        pltpu.VMEM((CHUNK, FDIM), jnp.float32),
        pltpu.VMEM((CHUNK,), jnp.float32),
        pltpu.VMEM((RPT,), jnp.float32),
        pltpu.SemaphoreType.DMA,
        pltpu.SemaphoreType.DMA,
        pltpu.SemaphoreType.DMA,
        pltpu.SemaphoreType.DMA,
        pltpu.SemaphoreType.DMA,
        pltpu.SemaphoreType.DMA,
    ],
)

_agg = pl.kernel(
    functools.partial(_sc_agg_body, False),
    out_type=[jax.ShapeDtypeStruct((NCORES, N_PAD, FDIM), jnp.float32)],
    mesh=_mesh,
    scratch_types=[
        pltpu.VMEM_SHARED((N_PAD, FDIM), jnp.float32),
        pltpu.VMEM((2, CHUNK), jnp.int32),
        pltpu.VMEM((2, CHUNK), jnp.int32),
        pltpu.VMEM((CHUNK, FDIM), jnp.float32),
        pltpu.VMEM((CHUNK, FDIM), jnp.float32),
        pltpu.SemaphoreType.DMA,
        pltpu.SemaphoreType.DMA,
        pltpu.SemaphoreType.DMA,
        pltpu.SemaphoreType.DMA,
    ],
)


def _dinv_from(pdeg_ref):
    d = pdeg_ref[0, 0]
    for k in range(1, NCORES):
        d = d + pdeg_ref[k, 0]
    return 1.0 / jnp.maximum(d, 1.0)


def _psum(p_ref):
    agg = p_ref[0]
    for k in range(1, NCORES):
        agg = agg + p_ref[k]
    return agg


def _c1_body(p_ref, pdeg_ref, w_ref, b_ref, out_ref):
    agg = _psum(p_ref) * _dinv_from(pdeg_ref)
    h = jnp.dot(agg, w_ref[...], preferred_element_type=jnp.float32) + b_ref[...]
    out_ref[...] = jnp.maximum(h, 0.0)


def _c2_body(p_ref, pdeg_ref, w_ref, b_ref, wout_ref, bout_ref, batch_ref,
             out_ref, s_acc, c_acc):
    i = pl.program_id(0)

    @pl.when(i == 0)
    def _():
        s_acc[...] = jnp.zeros_like(s_acc)
        c_acc[...] = jnp.zeros_like(c_acc)

    agg = _psum(p_ref) * _dinv_from(pdeg_ref)
    h2 = jnp.maximum(
        jnp.dot(agg, w_ref[...], preferred_element_type=jnp.float32) + b_ref[...],
        0.0)
    bb = batch_ref[0]
    gids = lax.broadcasted_iota(jnp.int32, (NGRAPH, BM), 0)
    oh = jnp.where(gids == bb, 1.0, 0.0)
    s_acc[...] += jnp.dot(oh, h2, preferred_element_type=jnp.float32)
    c_acc[...] += jnp.broadcast_to(
        jnp.sum(oh, axis=1, keepdims=True), (NGRAPH, FDIM))

    @pl.when(i == pl.num_programs(0) - 1)
    def _():
        cnt = c_acc[:, 0:NCLS]
        num = (jnp.dot(s_acc[...], wout_ref[...],
                       preferred_element_type=jnp.float32)
               + cnt * bout_ref[...])
        out_ref[...] = num / jnp.maximum(cnt, 1.0)


def _c1(p, pdeg, W, br):
    return pl.pallas_call(
        _c1_body,
        grid=(NBLK,),
        in_specs=[
            pl.BlockSpec((NCORES, BM, FDIM), lambda i: (0, i, 0)),
            pl.BlockSpec((NCORES, 1, BM, 1), lambda i: (0, i, 0, 0)),
            pl.BlockSpec((FDIM, FDIM), lambda i: (0, 0)),
            pl.BlockSpec((1, FDIM), lambda i: (0, 0)),
        ],
        out_specs=pl.BlockSpec((BM, FDIM), lambda i: (i, 0)),
        out_shape=jax.ShapeDtypeStruct((N_PAD, FDIM), jnp.float32),
    )(p, pdeg, W, br)


def _c2(p, pdeg, W, br, Wout, boutr, batch_r):
    return pl.pallas_call(
        _c2_body,
        grid=(NBLK,),
        in_specs=[
            pl.BlockSpec((NCORES, BM, FDIM), lambda i: (0, i, 0)),
            pl.BlockSpec((NCORES, 1, BM, 1), lambda i: (0, i, 0, 0)),
            pl.BlockSpec((FDIM, FDIM), lambda i: (0, 0)),
            pl.BlockSpec((1, FDIM), lambda i: (0, 0)),
            pl.BlockSpec((FDIM, NCLS), lambda i: (0, 0)),
            pl.BlockSpec((1, NCLS), lambda i: (0, 0)),
            pl.BlockSpec((1, 1, BM), lambda i: (i, 0, 0)),
        ],
        out_specs=pl.BlockSpec((NGRAPH, NCLS), lambda i: (0, 0)),
        out_shape=jax.ShapeDtypeStruct((NGRAPH, NCLS), jnp.float32),
        scratch_shapes=[
            pltpu.VMEM((NGRAPH, FDIM), jnp.float32),
            pltpu.VMEM((NGRAPH, FDIM), jnp.float32),
        ],
    )(p, pdeg, W, br, Wout, boutr, batch_r)


def kernel(x, edge_index, batch, W0, b0, W1, b1, Wout, bout):
    src = edge_index[0].astype(jnp.int32)
    dst = edge_index[1].astype(jnp.int32)
    npad = E_PAD - N_EDGES
    src_p = jnp.concatenate(
        [src, jnp.zeros((npad,), jnp.int32)]).reshape(TOTC, CHUNK)
    dst_p = jnp.concatenate(
        [dst, jnp.full((npad,), JUNK_ROW, jnp.int32)]).reshape(TOTC, CHUNK)
    eir = jnp.stack([src_p, dst_p], axis=1).reshape(2 * TOTC, CHUNK)
    batch_p = jnp.concatenate(
        [batch.astype(jnp.int32),
         jnp.full((N_PAD - N_NODES,), NGRAPH, jnp.int32)]).reshape(NBLK, 1, BM)
    p0, deg_flat = _agg_deg(x, eir)
    pdeg = deg_flat.reshape(NCORES, NBLK, BM, 1)
    h1 = _c1(p0, pdeg, W0, b0.reshape(1, FDIM))
    (p1,) = _agg(h1, eir)
    return _c2(p1, pdeg, W1, b1.reshape(1, FDIM), Wout,
               bout.reshape(1, NCLS), batch_p)


# speedup vs baseline: 1.0460x; 1.0023x over previous
import functools

import jax
import jax.numpy as jnp
from jax import lax
from jax.experimental import pallas as pl
from jax.experimental.pallas import tpu as pltpu
from jax.experimental.pallas import tpu_sc as plsc

N_NODES = 10000
N_PAD = 10240
FDIM = 128
NCLS = 40
NGRAPH = 64
N_EDGES = 320000
NCORES = 2
NSUB = 16
NW = NCORES * NSUB
CHUNK = 128
TOTC = 2560
E_PAD = TOTC * CHUNK
NC0 = 148
NC1 = 12
JUNK_ROW = N_PAD - 1
RPT = N_PAD // NSUB
BM = 1024
NBLK = N_PAD // BM

_mesh = plsc.VectorSubcoreMesh(core_axis_name="c", subcore_axis_name="s",
                               num_cores=NCORES)


def _fill2(ref, nrows, ncols, val):
    v = jnp.full((16,), val, jnp.float32)

    def row(i, carry):
        for k in range(ncols // 16):
            ref[i, pl.ds(k * 16, 16)] = v
        return carry

    lax.fori_loop(0, nrows, row, 0)


def _fill1(ref, n, val):
    v = jnp.full((16,), val, jnp.float32)

    def step(i, carry):
        ref[pl.ds(i * 16, 16)] = v
        return carry

    lax.fori_loop(0, n // 16, step, 0)


def _sc_agg_body(with_deg, table, eir, *rest):
    if with_deg:
        (out, dout, acc, degs, ib0, ib1, rows0, rows1, ones_v, zv,
         g0, g1, s0, s1, d0, d1) = rest
    else:
        (out, acc, ib0, ib1, rows0, rows1, g0, g1, s0, s1) = rest
        degs = dout = ones_v = zv = d0 = d1 = None
    ibufs = (ib0, ib1)
    rowss = (rows0, rows1)
    gsem = (g0, g1)
    ssem = (s0, s1)
    dsem = (d0, d1)
    c = lax.axis_index("c")
    s = lax.axis_index("s")
    base = s * RPT
    nchunk = jnp.where(c == 0, NC0, NC1)
    k0 = jnp.where(c == 0, s * NC0, NSUB * NC0 + s * NC1)

    def load_idx(j, b):
        pltpu.sync_copy(eir.at[pl.ds(2 * (k0 + j), 2)], ibufs[b])

    def gather_start(b):
        pltpu.async_copy(table.at[ibufs[b].at[0]], rowss[b], gsem[b])

    def gather_wait(b):
        pltpu.make_async_copy(
            table.at[ibufs[b].at[0]], rowss[b], gsem[b]).wait()

    def scatter_start(b):
        pltpu.async_copy(rowss[b], acc.at[ibufs[b].at[1]], ssem[b], add=True)
        if with_deg:
            pltpu.async_copy(ones_v, degs.at[ibufs[b].at[1]], dsem[b],
                             add=True)

    def scatter_wait(b):
        pltpu.make_async_copy(rowss[b], acc.at[ibufs[b].at[1]], ssem[b]).wait()
        if with_deg:
            pltpu.make_async_copy(
                ones_v, degs.at[ibufs[b].at[1]], dsem[b]).wait()

    _fill2(rows0, CHUNK, FDIM, 0.0)
    for k in range(RPT // CHUNK):
        pltpu.sync_copy(rows0, acc.at[pl.ds(base + k * CHUNK, CHUNK)])
    if with_deg:
        _fill1(zv, RPT, 0.0)
        pltpu.sync_copy(zv, degs.at[pl.ds(base, RPT)])
        _fill1(ones_v, CHUNK, 1.0)
    load_idx(0, 0)
    gather_start(0)
    load_idx(1, 1)
    gather_start(1)
    plsc.subcore_barrier()
    gather_wait(0)
    scatter_start(0)

    def step(j, cur):
        nxt = 1 - cur
        scatter_wait(nxt)
        load_idx(j + 1, nxt)
        gather_start(nxt)
        gather_wait(cur)
        scatter_start(cur)

    def pair(p, carry):
        step(2 * p + 1, 1)
        step(2 * p + 2, 0)
        return carry

    lax.fori_loop(0, (nchunk - 2) // 2, pair, 0)
    scatter_wait(0)
    gather_wait(1)
    scatter_start(1)
    scatter_wait(1)
    plsc.subcore_barrier()
    pltpu.sync_copy(acc.at[pl.ds(base, RPT)], out.at[c, pl.ds(base, RPT)])
    if with_deg:
        pltpu.sync_copy(degs.at[pl.ds(base, RPT)],
                        dout.at[pl.ds(c * N_PAD + base, RPT)])


_agg_deg = pl.kernel(
    functools.partial(_sc_agg_body, True),
    out_type=[
        jax.ShapeDtypeStruct((NCORES, N_PAD, FDIM), jnp.float32),
        jax.ShapeDtypeStruct((NCORES * N_PAD,), jnp.float32),
    ],
    mesh=_mesh,
    scratch_types=[
        pltpu.VMEM_SHARED((N_PAD, FDIM), jnp.float32),
        pltpu.VMEM_SHARED((N_PAD,), jnp.float32),
        pltpu.VMEM((2, CHUNK), jnp.int32),
        pltpu.VMEM((2, CHUNK), jnp.int32),
        pltpu.VMEM((CHUNK, FDIM), jnp.float32),
        pltpu.VMEM((CHUNK, FDIM), jnp.float32),
        pltpu.VMEM((CHUNK,), jnp.float32),
        pltpu.VMEM((RPT,), jnp.float32),
        pltpu.SemaphoreType.DMA,
        pltpu.SemaphoreType.DMA,
        pltpu.SemaphoreType.DMA,
        pltpu.SemaphoreType.DMA,
        pltpu.SemaphoreType.DMA,
        pltpu.SemaphoreType.DMA,
    ],
)

_agg = pl.kernel(
    functools.partial(_sc_agg_body, False),
    out_type=[jax.ShapeDtypeStruct((NCORES, N_PAD, FDIM), jnp.float32)],
    mesh=_mesh,
    scratch_types=[
        pltpu.VMEM_SHARED((N_PAD, FDIM), jnp.float32),
        pltpu.VMEM((2, CHUNK), jnp.int32),
        pltpu.VMEM((2, CHUNK), jnp.int32),
        pltpu.VMEM((CHUNK, FDIM), jnp.float32),
        pltpu.VMEM((CHUNK, FDIM), jnp.float32),
        pltpu.SemaphoreType.DMA,
        pltpu.SemaphoreType.DMA,
        pltpu.SemaphoreType.DMA,
        pltpu.SemaphoreType.DMA,
    ],
)


def _dinv_from(pdeg_ref):
    d = pdeg_ref[0, 0]
    for k in range(1, NCORES):
        d = d + pdeg_ref[k, 0]
    return 1.0 / jnp.maximum(d, 1.0)


def _psum(p_ref):
    agg = p_ref[0]
    for k in range(1, NCORES):
        agg = agg + p_ref[k]
    return agg


def _c1_body(p_ref, pdeg_ref, w_ref, b_ref, out_ref):
    agg = _psum(p_ref) * _dinv_from(pdeg_ref)
    h = jnp.dot(agg, w_ref[...], preferred_element_type=jnp.float32) + b_ref[...]
    out_ref[...] = jnp.maximum(h, 0.0)


def _c2_body(p_ref, pdeg_ref, w_ref, b_ref, wout_ref, bout_ref, batch_ref,
             out_ref, s_acc, c_acc):
    i = pl.program_id(0)

    @pl.when(i == 0)
    def _():
        s_acc[...] = jnp.zeros_like(s_acc)
        c_acc[...] = jnp.zeros_like(c_acc)

    agg = _psum(p_ref) * _dinv_from(pdeg_ref)
    h2 = jnp.maximum(
        jnp.dot(agg, w_ref[...], preferred_element_type=jnp.float32) + b_ref[...],
        0.0)
    bb = batch_ref[0]
    gids = lax.broadcasted_iota(jnp.int32, (NGRAPH, BM), 0)
    oh = jnp.where(gids == bb, 1.0, 0.0)
    s_acc[...] += jnp.dot(oh, h2, preferred_element_type=jnp.float32)
    c_acc[...] += jnp.broadcast_to(
        jnp.sum(oh, axis=1, keepdims=True), (NGRAPH, FDIM))

    @pl.when(i == pl.num_programs(0) - 1)
    def _():
        cnt = c_acc[:, 0:NCLS]
        num = (jnp.dot(s_acc[...], wout_ref[...],
                       preferred_element_type=jnp.float32)
               + cnt * bout_ref[...])
        out_ref[...] = num / jnp.maximum(cnt, 1.0)


def _c1(p, pdeg, W, br):
    return pl.pallas_call(
        _c1_body,
        grid=(NBLK,),
        in_specs=[
            pl.BlockSpec((NCORES, BM, FDIM), lambda i: (0, i, 0)),
            pl.BlockSpec((NCORES, 1, BM, 1), lambda i: (0, i, 0, 0)),
            pl.BlockSpec((FDIM, FDIM), lambda i: (0, 0)),
            pl.BlockSpec((1, FDIM), lambda i: (0, 0)),
        ],
        out_specs=pl.BlockSpec((BM, FDIM), lambda i: (i, 0)),
        out_shape=jax.ShapeDtypeStruct((N_PAD, FDIM), jnp.float32),
    )(p, pdeg, W, br)


def _c2(p, pdeg, W, br, Wout, boutr, batch_r):
    return pl.pallas_call(
        _c2_body,
        grid=(NBLK,),
        in_specs=[
            pl.BlockSpec((NCORES, BM, FDIM), lambda i: (0, i, 0)),
            pl.BlockSpec((NCORES, 1, BM, 1), lambda i: (0, i, 0, 0)),
            pl.BlockSpec((FDIM, FDIM), lambda i: (0, 0)),
            pl.BlockSpec((1, FDIM), lambda i: (0, 0)),
            pl.BlockSpec((FDIM, NCLS), lambda i: (0, 0)),
            pl.BlockSpec((1, NCLS), lambda i: (0, 0)),
            pl.BlockSpec((1, 1, BM), lambda i: (i, 0, 0)),
        ],
        out_specs=pl.BlockSpec((NGRAPH, NCLS), lambda i: (0, 0)),
        out_shape=jax.ShapeDtypeStruct((NGRAPH, NCLS), jnp.float32),
        scratch_shapes=[
            pltpu.VMEM((NGRAPH, FDIM), jnp.float32),
            pltpu.VMEM((NGRAPH, FDIM), jnp.float32),
        ],
    )(p, pdeg, W, br, Wout, boutr, batch_r)


def kernel(x, edge_index, batch, W0, b0, W1, b1, Wout, bout):
    src = edge_index[0].astype(jnp.int32)
    dst = edge_index[1].astype(jnp.int32)
    npad = E_PAD - N_EDGES
    src_p = jnp.concatenate(
        [src, jnp.zeros((npad,), jnp.int32)]).reshape(TOTC, CHUNK)
    dst_p = jnp.concatenate(
        [dst, jnp.full((npad,), JUNK_ROW, jnp.int32)]).reshape(TOTC, CHUNK)
    eir = jnp.stack([src_p, dst_p], axis=1).reshape(2 * TOTC, CHUNK)
    batch_p = jnp.concatenate(
        [batch.astype(jnp.int32),
         jnp.full((N_PAD - N_NODES,), NGRAPH, jnp.int32)]).reshape(NBLK, 1, BM)
    p0, deg_flat = _agg_deg(x, eir)
    pdeg = deg_flat.reshape(NCORES, NBLK, BM, 1)
    h1 = _c1(p0, pdeg, W0, b0.reshape(1, FDIM))
    (p1,) = _agg(h1, eir)
    return _c2(p1, pdeg, W1, b1.reshape(1, FDIM), Wout,
               bout.reshape(1, NCLS), batch_p)
